# baseline (speedup 1.0000x reference)
import functools

import jax
import jax.numpy as jnp
from jax import lax
from jax.experimental import pallas as pl
from jax.experimental.pallas import tpu as pltpu
from jax.experimental.pallas import tpu_sc as plsc

NC, NS = 2, 16
NW = NC * NS
CG = 2000
CS = 80
NB = 2000
EBF = 1600
F = 8

_LOG2E = 1.4426950408889634


def _sig(x):
    return 1.0 / (1.0 + jnp.exp2(x * (-_LOG2E)))


def _silu(x):
    return x * _sig(x)


def _proj_body(nf_ref, w_ref, *out_refs):
    nf = nf_ref[...]
    for r in range(2):
        p = jnp.dot(nf, w_ref[r], preferred_element_type=jnp.float32)
        for h in range(4):
            out_refs[4 * r + h][...] = p[:, 16 * h:16 * (h + 1)]


def _edge_body(gs0, gs1, gs2, gs3, gd0, gd1, gd2, gd3, ea_ref, ew_ref,
               bd16_ref, b16_ref, bdel_ref, w128_ref, b128_ref, wnl_ref,
               ea2_ref, feat_ref):
    def mm(x, w):
        return jnp.dot(x, w, preferred_element_type=jnp.float32)

    ea = ea_ref[...]
    ew = ew_ref[...]
    h1 = _silu(gs0[...] + gd0[...] + mm(ea, bd16_ref[0]) + b16_ref[0])
    g1 = _silu(gs1[...] + gd1[...] + mm(ea, bd16_ref[1]) + b16_ref[1])
    h2 = _silu(mm(h1, bd16_ref[2]) + b16_ref[2])
    g2 = _sig(mm(g1, bd16_ref[3]) + b16_ref[3])
    ea2 = ea + h2 * g2 * mm(ew, bdel_ref[...])
    ea2_ref[...] = ea2
    hn1 = _silu(gs2[...] + gd2[...] + mm(ea2, bd16_ref[4]) + b16_ref[4])
    gn1 = _silu(gs3[...] + gd3[...] + mm(ea2, bd16_ref[5]) + b16_ref[5])
    for k in range(F):
        hk = hn1[:, 16 * k:16 * (k + 1)]
        gk = gn1[:, 16 * k:16 * (k + 1)]
        ek = ew[:, 9 * k:9 * (k + 1)]
        hn2 = _silu(mm(hk, w128_ref[0]) + b128_ref[0])
        gn2 = _sig(mm(gk, w128_ref[1]) + b128_ref[1])
        feat_ref[k] = hn2 * gn2 * mm(ek, wnl_ref[...])


def _comb_body(nf_ref, p_ref, o_ref):
    o_ref[...] = nf_ref[...] + p_ref[0] + p_ref[1]


def kernel(node_features, edge_attr, edge_weights, edge_index, We1, be1, We2,
           be2, Wge1, bge1, Wge2, bge2, Wel, Wn1, bn1, Wn2, bn2, Wgn1, bgn1,
           Wgn2, bgn2, Wnl):
    N, DN = node_features.shape
    E, DE = edge_attr.shape
    DEG = edge_weights.shape[1]
    f32 = jnp.float32
    src = edge_index[0]
    dst = edge_index[1]
    EF = E // F

    wsrc = jnp.concatenate([We1[:DN], Wge1[:DN], Wn1[:DN], Wgn1[:DN]], axis=1)
    wdst = jnp.concatenate([We1[DN:2 * DN], Wge1[DN:2 * DN], Wn1[DN:2 * DN],
                            Wgn1[DN:2 * DN]], axis=1)
    wstack = jnp.stack([wsrc, wdst])
    eye8 = jnp.eye(F, dtype=f32)
    bd16 = jnp.stack([jnp.kron(eye8, W) for W in
                      (We1[2 * DN:], Wge1[2 * DN:], We2, Wge2,
                       Wn1[2 * DN:], Wgn1[2 * DN:])])
    b16 = jnp.stack([jnp.tile(b, F) for b in
                     (be1, bge1, be2, bge2, bn1, bgn1)])
    bdel = jnp.kron(eye8, Wel)
    w128 = jnp.stack([Wn2, Wgn2])
    b128 = jnp.stack([bn2, bgn2])
    zeros_nd = jnp.zeros((N, DN), f32)
    ea_f = edge_attr.reshape(EF, F * DE)
    ew_f = edge_weights.reshape(EF, F * DEG)
    src_perm = src.reshape(EF, F).transpose(1, 0).reshape(E)

    ptabs = pl.pallas_call(
        _proj_body,
        grid=(N // NB,),
        in_specs=[pl.BlockSpec((NB, DN), lambda i: (i, 0)),
                  pl.BlockSpec((2, DN, 64), lambda i: (0, 0, 0))],
        out_specs=[pl.BlockSpec((NB, 16), lambda i: (i, 0))] * 8,
        out_shape=[jax.ShapeDtypeStruct((N, 16), f32)] * 8,
    )(node_features, wstack)

    mesh = plsc.VectorSubcoreMesh(core_axis_name="c", subcore_axis_name="s")
    rows_g = E // 16
    nch_g = rows_g // CG
    @functools.partial(
        pl.kernel,
        out_type=[jax.ShapeDtypeStruct((E, 16), f32)] * 8,
        mesh=mesh,
        scratch_types=[pltpu.VMEM((CG,), jnp.int32)] +
                      [pltpu.VMEM((CG, 16), f32) for _ in range(4)] +
                      [pltpu.SemaphoreType.DMA],
        compiler_params=pltpu.CompilerParams(use_tc_tiling_on_sc=False),
    )
    def _gather_k(t0, t1, t2, t3, t4, t5, t6, t7, src_hbm, dst_hbm,
                  o0, o1, o2, o3, o4, o5, o6, o7,
                  idx_v, r0, r1, r2, r3, sem):
        cid = lax.axis_index("c")
        sid = lax.axis_index("s")
        wid = sid * NC + cid
        base = (wid % 16) * rows_g
        rbufs = (r0, r1, r2, r3)

        def run(tabs, idxarr, outs):
            def chunk(k, carry):
                off = base + k * CG
                pltpu.sync_copy(idxarr.at[pl.ds(off, CG)], idx_v)
                cps = [pltpu.async_copy(tabs[h].at[idx_v], rbufs[h], sem)
                       for h in range(4)]
                for cp in cps:
                    cp.wait()
                for h in range(4):
                    pltpu.sync_copy(rbufs[h], outs[h].at[pl.ds(off, CG)])
                return carry
            lax.fori_loop(0, nch_g, chunk, 0)

        @pl.when(wid < 16)
        def _():
            run((t0, t1, t2, t3), src_hbm, (o0, o1, o2, o3))

        @pl.when(wid >= 16)
        def _():
            run((t4, t5, t6, t7), dst_hbm, (o4, o5, o6, o7))

    g_heads = _gather_k(*ptabs, src, dst)
    gf = [g.reshape(EF, F * 16) for g in g_heads]

    full = lambda *s: pl.BlockSpec(s, lambda i: tuple(0 for _ in s))
    ea2_f, feat8 = pl.pallas_call(
        _edge_body,
        grid=(EF // EBF,),
        in_specs=[pl.BlockSpec((EBF, F * 16), lambda i: (i, 0))] * 8 +
                 [pl.BlockSpec((EBF, F * DE), lambda i: (i, 0)),
                  pl.BlockSpec((EBF, F * DEG), lambda i: (i, 0)),
                  full(6, 128, 128), full(6, 128), full(F * DEG, 128),
                  full(2, DE, DN), full(2, DN), full(DEG, DN)],
        out_specs=[pl.BlockSpec((EBF, F * DE), lambda i: (i, 0)),
                   pl.BlockSpec((F, EBF, DN), lambda i: (0, i, 0))],
        out_shape=[jax.ShapeDtypeStruct((EF, F * DE), f32),
                   jax.ShapeDtypeStruct((F, EF, DN), f32)],
        compiler_params=pltpu.CompilerParams(
            dimension_semantics=("arbitrary",),
            allow_input_fusion=[False] * 8 + [True, True] + [False] * 6),
    )(gf[0], gf[1], gf[2], gf[3], gf[4], gf[5], gf[6], gf[7], ea_f, ew_f,
      bd16, b16, bdel, w128, b128, Wnl)

    ea2 = ea2_f.reshape(E, DE)
    feat = feat8.reshape(E, DN)

    rows_s = E // NW
    nch_s = rows_s // CS
    NT = N // NS

    src2 = src_perm.reshape(E // CS, CS)
    rpw = rows_s // CS

    @functools.partial(
        pl.kernel,
        out_type=jax.ShapeDtypeStruct((2, N, DN), f32),
        mesh=mesh,
        scratch_types=[pltpu.VMEM((rows_s // CS, CS), jnp.int32),
                       pltpu.VMEM((CS, DN), f32),
                       pltpu.VMEM((CS, DN), f32),
                       pltpu.VMEM((CS, DN), f32),
                       pltpu.SemaphoreType.DMA,
                       pltpu.SemaphoreType.DMA,
                       pltpu.SemaphoreType.DMA,
                       pltpu.VMEM_SHARED((N, DN), f32)],
        compiler_params=pltpu.CompilerParams(use_tc_tiling_on_sc=False),
    )
    def _scatter_k(feat_hbm, src_hbm, z_hbm, out_hbm, idx_all, rows0,
                   rows1, rows2, sem0, sem1, sem2, acc):
        cid = lax.axis_index("c")
        sid = lax.axis_index("s")
        wid = sid * NC + cid
        pltpu.sync_copy(src_hbm.at[pl.ds(wid * rpw, rpw)], idx_all)
        pltpu.sync_copy(z_hbm.at[pl.ds(sid * NT, NT)],
                        acc.at[pl.ds(sid * NT, NT)])
        plsc.subcore_barrier()
        base = wid * rows_s
        bufs = ((rows0, sem0), (rows1, sem1), (rows2, sem2))
        NBUF = 3

        def fire(k, b):
            rows_v, sem = bufs[b]
            pltpu.async_copy(feat_hbm.at[pl.ds(base + k * CS, CS)], rows_v,
                             sem)

        def drain_and_add(k, b):
            rows_v, sem = bufs[b]
            pltpu.make_async_copy(feat_hbm.at[pl.ds(base, CS)], rows_v,
                                  sem).wait()
            pltpu.sync_copy(rows_v, acc.at[idx_all.at[k]], add=True)

        for b0 in range(NBUF):
            fire(b0, b0)

        def group(t, carry):
            for b in range(NBUF):
                k = NBUF * t + b
                drain_and_add(k, b)

                @pl.when(k + NBUF < nch_s)
                def _():
                    fire(k + NBUF, b)
            return carry
        lax.fori_loop(0, nch_s // NBUF, group, 0)
        for k in range(nch_s - nch_s % NBUF, nch_s):
            drain_and_add(k, k % NBUF)
        plsc.subcore_barrier()
        pltpu.sync_copy(acc.at[pl.ds(sid * NT, NT)],
                        out_hbm.at[cid, pl.ds(sid * NT, NT)])

    parts = _scatter_k(feat, src2, zeros_nd)

    node2 = pl.pallas_call(
        _comb_body,
        grid=(N // NB,),
        in_specs=[pl.BlockSpec((NB, DN), lambda i: (i, 0)),
                  pl.BlockSpec((2, NB, DN), lambda i: (0, i, 0))],
        out_specs=pl.BlockSpec((NB, DN), lambda i: (i, 0)),
        out_shape=jax.ShapeDtypeStruct((N, DN), f32),
    )(node_features, parts)

    return (node2, ea2)


# speedup vs baseline: 1.0028x; 1.0028x over previous
import functools

import jax
import jax.numpy as jnp
from jax import lax
from jax.experimental import pallas as pl
from jax.experimental.pallas import tpu as pltpu
from jax.experimental.pallas import tpu_sc as plsc

NC, NS = 2, 16
NW = NC * NS
CG = 2000
CS = 80
NB = 2000
EBF = 1600
F = 8

_LOG2E = 1.4426950408889634


def _sig(x):
    return 1.0 / (1.0 + jnp.exp2(x * (-_LOG2E)))


def _silu(x):
    return x * _sig(x)


def _proj_body(nf_ref, w_ref, *out_refs):
    nf = nf_ref[...]
    for r in range(2):
        p = jnp.dot(nf, w_ref[r], preferred_element_type=jnp.float32)
        for h in range(4):
            out_refs[4 * r + h][...] = p[:, 16 * h:16 * (h + 1)]


def _edge_body(gs0, gs1, gs2, gs3, gd0, gd1, gd2, gd3, ea_ref, ew_ref,
               bd16_ref, b16_ref, bdel_ref, w128_ref, b128_ref, wnl_ref,
               ea2_ref, feat_ref):
    def mm(x, w):
        return jnp.dot(x, w, preferred_element_type=jnp.float32)

    ea = ea_ref[...]
    ew = ew_ref[...]
    h1 = _silu(gs0[...] + gd0[...] + mm(ea, bd16_ref[0]) + b16_ref[0])
    g1 = _silu(gs1[...] + gd1[...] + mm(ea, bd16_ref[1]) + b16_ref[1])
    h2 = _silu(mm(h1, bd16_ref[2]) + b16_ref[2])
    g2 = _sig(mm(g1, bd16_ref[3]) + b16_ref[3])
    ea2 = ea + h2 * g2 * mm(ew, bdel_ref[...])
    ea2_ref[...] = ea2
    hn1 = _silu(gs2[...] + gd2[...] + mm(ea2, bd16_ref[4]) + b16_ref[4])
    gn1 = _silu(gs3[...] + gd3[...] + mm(ea2, bd16_ref[5]) + b16_ref[5])
    for k in range(F):
        hk = hn1[:, 16 * k:16 * (k + 1)]
        gk = gn1[:, 16 * k:16 * (k + 1)]
        ek = ew[:, 9 * k:9 * (k + 1)]
        hn2 = _silu(mm(hk, w128_ref[0]) + b128_ref[0])
        gn2 = _sig(mm(gk, w128_ref[1]) + b128_ref[1])
        feat_ref[k] = hn2 * gn2 * mm(ek, wnl_ref[...])


def _comb_body(nf_ref, p_ref, o_ref):
    o_ref[...] = nf_ref[...] + p_ref[0] + p_ref[1]


def kernel(node_features, edge_attr, edge_weights, edge_index, We1, be1, We2,
           be2, Wge1, bge1, Wge2, bge2, Wel, Wn1, bn1, Wn2, bn2, Wgn1, bgn1,
           Wgn2, bgn2, Wnl):
    N, DN = node_features.shape
    E, DE = edge_attr.shape
    DEG = edge_weights.shape[1]
    f32 = jnp.float32
    src = edge_index[0]
    dst = edge_index[1]
    EF = E // F

    wsrc = jnp.concatenate([We1[:DN], Wge1[:DN], Wn1[:DN], Wgn1[:DN]], axis=1)
    wdst = jnp.concatenate([We1[DN:2 * DN], Wge1[DN:2 * DN], Wn1[DN:2 * DN],
                            Wgn1[DN:2 * DN]], axis=1)
    wstack = jnp.stack([wsrc, wdst])
    eye8 = jnp.eye(F, dtype=f32)
    bd16 = jnp.stack([jnp.kron(eye8, W) for W in
                      (We1[2 * DN:], Wge1[2 * DN:], We2, Wge2,
                       Wn1[2 * DN:], Wgn1[2 * DN:])])
    b16 = jnp.stack([jnp.tile(b, F) for b in
                     (be1, bge1, be2, bge2, bn1, bgn1)])
    bdel = jnp.kron(eye8, Wel)
    w128 = jnp.stack([Wn2, Wgn2])
    b128 = jnp.stack([bn2, bgn2])
    zeros_nd = jnp.zeros((N, DN), f32)
    ea_f = edge_attr.reshape(EF, F * DE)
    ew_f = edge_weights.reshape(EF, F * DEG)
    src_perm = src.reshape(EF, F).transpose(1, 0).reshape(E)

    ptabs = pl.pallas_call(
        _proj_body,
        grid=(N // NB,),
        in_specs=[pl.BlockSpec((NB, DN), lambda i: (i, 0)),
                  pl.BlockSpec((2, DN, 64), lambda i: (0, 0, 0))],
        out_specs=[pl.BlockSpec((NB, 16), lambda i: (i, 0))] * 8,
        out_shape=[jax.ShapeDtypeStruct((N, 16), f32)] * 8,
    )(node_features, wstack)

    mesh = plsc.VectorSubcoreMesh(core_axis_name="c", subcore_axis_name="s")
    rows_g = E // 16
    nch_g = rows_g // CG
    @functools.partial(
        pl.kernel,
        out_type=[jax.ShapeDtypeStruct((E, 16), f32)] * 8,
        mesh=mesh,
        scratch_types=[pltpu.VMEM((CG,), jnp.int32)] +
                      [pltpu.VMEM((CG, 16), f32) for _ in range(4)] +
                      [pltpu.SemaphoreType.DMA],
        compiler_params=pltpu.CompilerParams(use_tc_tiling_on_sc=False),
    )
    def _gather_k(t0, t1, t2, t3, t4, t5, t6, t7, src_hbm, dst_hbm,
                  o0, o1, o2, o3, o4, o5, o6, o7,
                  idx_v, r0, r1, r2, r3, sem):
        cid = lax.axis_index("c")
        sid = lax.axis_index("s")
        wid = sid * NC + cid
        base = (wid % 16) * rows_g
        rbufs = (r0, r1, r2, r3)

        def run(tabs, idxarr, outs):
            def chunk(k, carry):
                off = base + k * CG
                pltpu.sync_copy(idxarr.at[pl.ds(off, CG)], idx_v)
                cps = [pltpu.async_copy(tabs[h].at[idx_v], rbufs[h], sem)
                       for h in range(4)]
                for cp in cps:
                    cp.wait()
                for h in range(4):
                    pltpu.sync_copy(rbufs[h], outs[h].at[pl.ds(off, CG)])
                return carry
            lax.fori_loop(0, nch_g, chunk, 0)

        @pl.when(wid < 16)
        def _():
            run((t0, t1, t2, t3), src_hbm, (o0, o1, o2, o3))

        @pl.when(wid >= 16)
        def _():
            run((t4, t5, t6, t7), dst_hbm, (o4, o5, o6, o7))

    g_heads = _gather_k(*ptabs, src, dst)
    gf = [g.reshape(EF, F * 16) for g in g_heads]

    full = lambda *s: pl.BlockSpec(s, lambda i: tuple(0 for _ in s))
    ea2_f, feat8 = pl.pallas_call(
        _edge_body,
        grid=(EF // EBF,),
        in_specs=[pl.BlockSpec((EBF, F * 16), lambda i: (i, 0))] * 8 +
                 [pl.BlockSpec((EBF, F * DE), lambda i: (i, 0)),
                  pl.BlockSpec((EBF, F * DEG), lambda i: (i, 0)),
                  full(6, 128, 128), full(6, 128), full(F * DEG, 128),
                  full(2, DE, DN), full(2, DN), full(DEG, DN)],
        out_specs=[pl.BlockSpec((EBF, F * DE), lambda i: (i, 0)),
                   pl.BlockSpec((F, EBF, DN), lambda i: (0, i, 0))],
        out_shape=[jax.ShapeDtypeStruct((EF, F * DE), f32),
                   jax.ShapeDtypeStruct((F, EF, DN), f32)],
        compiler_params=pltpu.CompilerParams(
            dimension_semantics=("arbitrary",)),
    )(gf[0], gf[1], gf[2], gf[3], gf[4], gf[5], gf[6], gf[7], ea_f, ew_f,
      bd16, b16, bdel, w128, b128, Wnl)

    ea2 = ea2_f.reshape(E, DE)
    feat = feat8.reshape(E, DN)

    rows_s = E // NW
    nch_s = rows_s // CS
    NT = N // NS

    src2 = src_perm.reshape(E // CS, CS)
    rpw = rows_s // CS

    @functools.partial(
        pl.kernel,
        out_type=jax.ShapeDtypeStruct((2, N, DN), f32),
        mesh=mesh,
        scratch_types=[pltpu.VMEM((rows_s // CS, CS), jnp.int32),
                       pltpu.VMEM((CS, DN), f32),
                       pltpu.VMEM((CS, DN), f32),
                       pltpu.VMEM((CS, DN), f32),
                       pltpu.SemaphoreType.DMA,
                       pltpu.SemaphoreType.DMA,
                       pltpu.SemaphoreType.DMA,
                       pltpu.VMEM_SHARED((N, DN), f32)],
        compiler_params=pltpu.CompilerParams(use_tc_tiling_on_sc=False),
    )
    def _scatter_k(feat_hbm, src_hbm, z_hbm, out_hbm, idx_all, rows0,
                   rows1, rows2, sem0, sem1, sem2, acc):
        cid = lax.axis_index("c")
        sid = lax.axis_index("s")
        wid = sid * NC + cid
        pltpu.sync_copy(src_hbm.at[pl.ds(wid * rpw, rpw)], idx_all)
        pltpu.sync_copy(z_hbm.at[pl.ds(sid * NT, NT)],
                        acc.at[pl.ds(sid * NT, NT)])
        plsc.subcore_barrier()
        base = wid * rows_s
        bufs = ((rows0, sem0), (rows1, sem1), (rows2, sem2))
        NBUF = 3

        def fire(k, b):
            rows_v, sem = bufs[b]
            pltpu.async_copy(feat_hbm.at[pl.ds(base + k * CS, CS)], rows_v,
                             sem)

        def drain_and_add(k, b):
            rows_v, sem = bufs[b]
            pltpu.make_async_copy(feat_hbm.at[pl.ds(base, CS)], rows_v,
                                  sem).wait()
            pltpu.sync_copy(rows_v, acc.at[idx_all.at[k]], add=True)

        for b0 in range(NBUF):
            fire(b0, b0)

        def group(t, carry):
            for b in range(NBUF):
                k = NBUF * t + b
                drain_and_add(k, b)

                @pl.when(k + NBUF < nch_s)
                def _():
                    fire(k + NBUF, b)
            return carry
        lax.fori_loop(0, nch_s // NBUF, group, 0)
        for k in range(nch_s - nch_s % NBUF, nch_s):
            drain_and_add(k, k % NBUF)
        plsc.subcore_barrier()
        pltpu.sync_copy(acc.at[pl.ds(sid * NT, NT)],
                        out_hbm.at[cid, pl.ds(sid * NT, NT)])

    parts = _scatter_k(feat, src2, zeros_nd)

    node2 = pl.pallas_call(
        _comb_body,
        grid=(N // NB,),
        in_specs=[pl.BlockSpec((NB, DN), lambda i: (i, 0)),
                  pl.BlockSpec((2, NB, DN), lambda i: (0, i, 0))],
        out_specs=pl.BlockSpec((NB, DN), lambda i: (i, 0)),
        out_shape=jax.ShapeDtypeStruct((N, DN), f32),
    )(node_features, parts)

    return (node2, ea2)
